# Initial kernel scaffold; baseline (speedup 1.0000x reference)
import jax
import jax.numpy as jnp
from jax.experimental import pallas as pl


def kernel(x, conv1_w, bn1_gamma, bn1_beta, bn1_mean, bn1_var, conv2_w, bn2_gamma, bn2_beta, bn2_mean, bn2_var, down_w, bn_down_gamma, bn_down_beta, bn_down_mean, bn_down_var):
    raise NotImplementedError("write your pallas kernel here")


# speedup vs baseline: 1.4586x; 1.4586x over previous
import jax
import jax.numpy as jnp
from jax.experimental import pallas as pl
from jax.experimental.pallas import tpu as pltpu

_EPS = 1e-5


def _fold(gamma, beta, mean, var):
    scale = gamma / jnp.sqrt(var + _EPS)
    bias = beta - mean * scale
    return scale.astype(jnp.float32), bias.astype(jnp.float32)


def _fused_block_kernel(xph_ref, wf_ref, sA_ref, bA_ref, w2_ref, s2_ref,
                        b2_ref, out_ref, p1_ref, yp_ref, p2_ref):
    Hp = xph_ref.shape[1] // 4
    Wo = out_ref.shape[2]
    Ho = out_ref.shape[1]
    Cin = xph_ref.shape[3]
    Cout = out_ref.shape[3]

    for kh in range(3):
        for kw in range(3):
            ph = 2 * (kh % 2) + (kw % 2)
            dh, dw = kh // 2, kw // 2
            win = xph_ref[0, pl.ds(ph * Hp + dh, Ho), dw:dw + Wo, :]
            c0 = (kh * 3 + kw) * Cin
            p1_ref[:, c0:c0 + Cin] = win.reshape(Ho * Wo, Cin)

    y = jnp.dot(p1_ref[...], wf_ref[...], preferred_element_type=jnp.float32)
    y = y * sA_ref[...] + bA_ref[...]
    ident = y[:, Cout:]
    main = jnp.maximum(y[:, :Cout], 0.0).astype(jnp.bfloat16)

    yp_ref[...] = jnp.zeros_like(yp_ref)
    yp_ref[1:Ho + 1, 1:Wo + 1, :] = main.reshape(Ho, Wo, Cout)

    for kh in range(3):
        for kw in range(3):
            win = yp_ref[kh:kh + Ho, kw:kw + Wo, :]
            c0 = (kh * 3 + kw) * Cout
            p2_ref[:, c0:c0 + Cout] = win.reshape(Ho * Wo, Cout)

    y2 = jnp.dot(p2_ref[...], w2_ref[...], preferred_element_type=jnp.float32)
    y2 = y2 * s2_ref[...] + b2_ref[...] + ident
    out_ref[0] = jnp.maximum(y2, 0.0).reshape(Ho, Wo, Cout)


def kernel(x, conv1_w, bn1_gamma, bn1_beta, bn1_mean, bn1_var, conv2_w,
           bn2_gamma, bn2_beta, bn2_mean, bn2_var, down_w, bn_down_gamma,
           bn_down_beta, bn_down_mean, bn_down_var):
    B, Cin, H, W = x.shape
    Cout = conv1_w.shape[0]
    Ho, Wo = H // 2, W // 2

    xn = jnp.transpose(x, (0, 2, 3, 1)).astype(jnp.float32)
    xpad = jnp.pad(xn, ((0, 0), (1, 1), (1, 1), (0, 0))).astype(jnp.bfloat16)
    xph = xpad.reshape(B, Ho + 1, 2, Wo + 1, 2, Cin)
    xph = jnp.transpose(xph, (0, 2, 4, 1, 3, 5))
    xph = xph.reshape(B, 4 * (Ho + 1), Wo + 1, Cin)

    w1 = jnp.transpose(conv1_w, (2, 3, 1, 0)).reshape(9 * Cin, Cout)
    s1, b1 = _fold(bn1_gamma, bn1_beta, bn1_mean, bn1_var)
    wd = jnp.transpose(down_w[:, :, 0, 0], (1, 0))
    sd, bd = _fold(bn_down_gamma, bn_down_beta, bn_down_mean, bn_down_var)
    w2m = jnp.transpose(conv2_w, (2, 3, 1, 0)).reshape(9 * Cout, Cout)
    s2, b2 = _fold(bn2_gamma, bn2_beta, bn2_mean, bn2_var)

    ctr = (1 * 3 + 1) * Cin
    wf = jnp.zeros((9 * Cin, 2 * Cout), jnp.float32)
    wf = wf.at[:, :Cout].set(w1)
    wf = wf.at[ctr:ctr + Cin, Cout:].set(wd)

    wf = wf.astype(jnp.bfloat16)
    w2m = w2m.astype(jnp.bfloat16)
    sA = jnp.concatenate([s1, sd])[None, :]
    bA = jnp.concatenate([b1, bd])[None, :]
    s2 = s2[None, :]
    b2 = b2[None, :]

    flops = 2 * B * Ho * Wo * Cout * (9 * Cin + Cin + 9 * Cout)
    bytes_acc = 2 * xph.size + 2 * wf.size + 2 * w2m.size + \
        4 * B * Ho * Wo * Cout

    out = pl.pallas_call(
        _fused_block_kernel,
        out_shape=jax.ShapeDtypeStruct((B, Ho, Wo, Cout), jnp.float32),
        grid=(B,),
        in_specs=[
            pl.BlockSpec((1, 4 * (Ho + 1), Wo + 1, Cin),
                         lambda b: (b, 0, 0, 0)),
            pl.BlockSpec((9 * Cin, 2 * Cout), lambda b: (0, 0)),
            pl.BlockSpec((1, 2 * Cout), lambda b: (0, 0)),
            pl.BlockSpec((1, 2 * Cout), lambda b: (0, 0)),
            pl.BlockSpec((9 * Cout, Cout), lambda b: (0, 0)),
            pl.BlockSpec((1, Cout), lambda b: (0, 0)),
            pl.BlockSpec((1, Cout), lambda b: (0, 0)),
        ],
        out_specs=pl.BlockSpec((1, Ho, Wo, Cout), lambda b: (b, 0, 0, 0)),
        scratch_shapes=[
            pltpu.VMEM((Ho * Wo, 9 * Cin), jnp.bfloat16),
            pltpu.VMEM((Ho + 2, Wo + 2, Cout), jnp.bfloat16),
            pltpu.VMEM((Ho * Wo, 9 * Cout), jnp.bfloat16),
        ],
        compiler_params=pltpu.CompilerParams(
            dimension_semantics=("parallel",),
            vmem_limit_bytes=64 * 1024 * 1024),
        cost_estimate=pl.CostEstimate(flops=flops, transcendentals=0,
                                      bytes_accessed=bytes_acc),
    )(xph, wf, sA, bA, w2m, s2, b2)
    return jnp.transpose(out, (0, 3, 1, 2))
